# Initial kernel scaffold; baseline (speedup 1.0000x reference)
import jax
import jax.numpy as jnp
from jax.experimental import pallas as pl


def kernel(x, edge_index, W1l, b1l, W1r, W2l, b2l, W2r, W3l, b3l, W3r, gamma, beta):
    raise NotImplementedError("write your pallas kernel here")


# speedup vs baseline: 4.0297x; 4.0297x over previous
import functools

import jax
import jax.numpy as jnp
from jax import lax
from jax.experimental import pallas as pl
from jax.experimental.pallas import tpu as pltpu
from jax.experimental.pallas import tpu_sc as plsc

N = 10000
E = 320000
NC, NS = 2, 16
NW = NC * NS
CH = 128
KG = 16
NG = 5
NCH = KG * NG
EPT = CH * NCH
N_ACC = 10112
RPT = N_ACC // NS


def _make_seg_sum(d, compute_deg):
    mesh = plsc.VectorSubcoreMesh(core_axis_name="c", subcore_axis_name="s")
    out_type = [jax.ShapeDtypeStruct((NC, N_ACC, d), jnp.float32)]
    scratch = [
        pltpu.VMEM((KG, CH), jnp.int32),
        pltpu.VMEM((KG, CH), jnp.int32),
        pltpu.VMEM((CH, d), jnp.float32),
        pltpu.VMEM_SHARED((N_ACC, d), jnp.float32),
    ]
    if compute_deg:
        out_type.append(jax.ShapeDtypeStruct((NC, N_ACC, 16), jnp.float32))
        scratch += [
            pltpu.VMEM((CH, 16), jnp.float32),
            pltpu.VMEM_SHARED((N_ACC, 16), jnp.float32),
        ]

    def body(y, srcs, dsts, zeros, *rest):
        if compute_deg:
            (zeros16, ones_in, out, dout,
             src_v, dst_v, rows, acc, ones_v, dacc) = rest
        else:
            out, src_v, dst_v, rows, acc = rest
        cid = lax.axis_index("c")
        sid = lax.axis_index("s")
        wid = cid * NS + sid
        r0 = sid * RPT
        pltpu.sync_copy(zeros.at[pl.ds(r0, RPT)], acc.at[pl.ds(r0, RPT)])
        if compute_deg:
            pltpu.sync_copy(zeros16.at[pl.ds(r0, RPT)], dacc.at[pl.ds(r0, RPT)])
            pltpu.sync_copy(ones_in, ones_v)
        plsc.subcore_barrier()

        def group(g, carry):
            pltpu.sync_copy(srcs.at[wid, pl.ds(g * KG, KG)], src_v)
            pltpu.sync_copy(dsts.at[wid, pl.ds(g * KG, KG)], dst_v)
            for k in range(KG):
                pltpu.sync_copy(y.at[src_v.at[k]], rows)
                pltpu.sync_copy(rows, acc.at[dst_v.at[k]], add=True)
                if compute_deg:
                    pltpu.sync_copy(ones_v, dacc.at[dst_v.at[k]], add=True)
            return carry

        lax.fori_loop(0, NG, group, 0)
        plsc.subcore_barrier()
        pltpu.sync_copy(acc.at[pl.ds(r0, RPT)], out.at[cid, pl.ds(r0, RPT)])
        if compute_deg:
            pltpu.sync_copy(dacc.at[pl.ds(r0, RPT)], dout.at[cid, pl.ds(r0, RPT)])

    return pl.kernel(body, out_type=tuple(out_type), mesh=mesh,
                     scratch_types=scratch,
                     compiler_params=pltpu.CompilerParams(
                         use_tc_tiling_on_sc=False))


def _mm2_body(x_ref, wl_ref, wr_ref, y_ref, z_ref):
    x = x_ref[...]
    y_ref[...] = jnp.dot(x, wl_ref[...], preferred_element_type=jnp.float32)
    z_ref[...] = jnp.dot(x, wr_ref[...], preferred_element_type=jnp.float32)


def _mid_body(s_ref, d_ref, z_ref, b_ref, g_ref, be_ref, wl_ref, wr_ref,
              y_ref, zo_ref):
    s = s_ref[0] + s_ref[1]
    deg = d_ref[0, :, 0:1] + d_ref[1, :, 0:1]
    t = s / jnp.maximum(deg, 1.0) + b_ref[...] + z_ref[...]
    m = jnp.mean(t, axis=-1, keepdims=True)
    v = jnp.mean((t - m) ** 2, axis=-1, keepdims=True)
    h = (t - m) / jnp.sqrt(v + 1e-5) * g_ref[...] + be_ref[...]
    h = jnp.maximum(h, 0.0)
    y_ref[...] = jnp.dot(h, wl_ref[...], preferred_element_type=jnp.float32)
    zo_ref[...] = jnp.dot(h, wr_ref[...], preferred_element_type=jnp.float32)


def _fin_body(s_ref, d_ref, z_ref, b_ref, o_ref):
    s = s_ref[0] + s_ref[1]
    deg = d_ref[0, :, 0:1] + d_ref[1, :, 0:1]
    o_ref[...] = s / jnp.maximum(deg, 1.0) + b_ref[...] + z_ref[...]


BLK = 1000
GRID = N // BLK


def _tc_mm2(x, wl, wr):
    din, dout = wl.shape
    return pl.pallas_call(
        _mm2_body,
        grid=(GRID,),
        in_specs=[
            pl.BlockSpec((BLK, din), lambda i: (i, 0)),
            pl.BlockSpec((din, dout), lambda i: (0, 0)),
            pl.BlockSpec((din, dout), lambda i: (0, 0)),
        ],
        out_specs=[
            pl.BlockSpec((BLK, dout), lambda i: (i, 0)),
            pl.BlockSpec((BLK, dout), lambda i: (i, 0)),
        ],
        out_shape=[jax.ShapeDtypeStruct((N, dout), jnp.float32)] * 2,
    )(x, wl, wr)


def _tc_mid(s, dg, z, b, g, be, wl, wr):
    din, dout = wl.shape
    return pl.pallas_call(
        _mid_body,
        grid=(GRID,),
        in_specs=[
            pl.BlockSpec((NC, BLK, din), lambda i: (0, i, 0)),
            pl.BlockSpec((NC, BLK, 16), lambda i: (0, i, 0)),
            pl.BlockSpec((BLK, din), lambda i: (i, 0)),
            pl.BlockSpec((1, din), lambda i: (0, 0)),
            pl.BlockSpec((1, din), lambda i: (0, 0)),
            pl.BlockSpec((1, din), lambda i: (0, 0)),
            pl.BlockSpec((din, dout), lambda i: (0, 0)),
            pl.BlockSpec((din, dout), lambda i: (0, 0)),
        ],
        out_specs=[
            pl.BlockSpec((BLK, dout), lambda i: (i, 0)),
            pl.BlockSpec((BLK, dout), lambda i: (i, 0)),
        ],
        out_shape=[jax.ShapeDtypeStruct((N, dout), jnp.float32)] * 2,
    )(s, dg, z, b, g, be, wl, wr)


def _tc_fin(s, dg, z, b):
    dout = z.shape[1]
    return pl.pallas_call(
        _fin_body,
        grid=(GRID,),
        in_specs=[
            pl.BlockSpec((NC, BLK, dout), lambda i: (0, i, 0)),
            pl.BlockSpec((NC, BLK, 16), lambda i: (0, i, 0)),
            pl.BlockSpec((BLK, dout), lambda i: (i, 0)),
            pl.BlockSpec((1, dout), lambda i: (0, 0)),
        ],
        out_specs=pl.BlockSpec((BLK, dout), lambda i: (i, 0)),
        out_shape=jax.ShapeDtypeStruct((N, dout), jnp.float32),
    )(s, dg, z, b)


_seg_sum_deg_128 = _make_seg_sum(128, True)
_seg_sum_128 = _make_seg_sum(128, False)
_seg_sum_64 = _make_seg_sum(64, False)


def kernel(x, edge_index, W1l, b1l, W1r, W2l, b2l, W2r, W3l, b3l, W3r,
           gamma, beta):
    src = edge_index[0]
    dst = edge_index[1]
    pad = NW * EPT - E
    src_p = jnp.concatenate([src, jnp.zeros((pad,), jnp.int32)])
    dst_p = jnp.concatenate([dst, jnp.full((pad,), N, jnp.int32)])
    src_p = src_p.reshape(NW, NCH, CH)
    dst_p = dst_p.reshape(NW, NCH, CH)
    z128 = jnp.zeros((N_ACC, 128), jnp.float32)
    z64 = jnp.zeros((N_ACC, 64), jnp.float32)
    z16 = jnp.zeros((N_ACC, 16), jnp.float32)
    ones16 = jnp.ones((CH, 16), jnp.float32)
    b1 = b1l.reshape(1, -1)
    b2 = b2l.reshape(1, -1)
    b3 = b3l.reshape(1, -1)
    g = gamma.reshape(1, -1)
    be = beta.reshape(1, -1)

    y1, zz1 = _tc_mm2(x, W1l, W1r)
    s1, dg = _seg_sum_deg_128(y1, src_p, dst_p, z128, z16, ones16)
    y2, zz2 = _tc_mid(s1, dg, zz1, b1, g, be, W2l, W2r)
    s2 = _seg_sum_128(y2, src_p, dst_p, z128)[0]
    y3, zz3 = _tc_mid(s2, dg, zz2, b2, g, be, W3l, W3r)
    s3 = _seg_sum_64(y3, src_p, dst_p, z64)[0]
    return _tc_fin(s3, dg, zz3, b3)


# speedup vs baseline: 5.5733x; 1.3831x over previous
import functools

import jax
import jax.numpy as jnp
from jax import lax
from jax.experimental import pallas as pl
from jax.experimental.pallas import tpu as pltpu
from jax.experimental.pallas import tpu_sc as plsc

N = 10000
E = 320000
NC, NS = 2, 16
NW = NC * NS
CH = 128
NCH_E = 80
NCH_C = 160
E_PAD = NW * NCH_E * CH
N_ACC = 10112
RPT = N_ACC // NS
B = 4

_SC_PARAMS = pltpu.CompilerParams(use_tc_tiling_on_sc=False)


def _seg_body(nch, col_split, y, srcs, dsts, zeros, out,
              src_v, dst_v, rows, acc, *sems):
    gsem = sems[:B]
    ssem = sems[B:]
    cid = lax.axis_index("c")
    sid = lax.axis_index("s")
    r0 = sid * RPT
    pltpu.sync_copy(zeros.at[pl.ds(r0, RPT)], acc.at[pl.ds(r0, RPT)])
    if col_split:
        table = y.at[cid]
        pltpu.sync_copy(srcs.at[sid], src_v)
        pltpu.sync_copy(dsts.at[sid], dst_v)
    else:
        table = y
        wid = cid * NS + sid
        pltpu.sync_copy(srcs.at[wid], src_v)
        pltpu.sync_copy(dsts.at[wid], dst_v)
    plsc.subcore_barrier()

    def gather(c, slot):
        pltpu.async_copy(table.at[src_v.at[c]], rows.at[slot], gsem[slot])

    def wait_gather(c, slot):
        pltpu.make_async_copy(table.at[src_v.at[c]], rows.at[slot],
                              gsem[slot]).wait()

    def scatter(c, slot):
        pltpu.async_copy(rows.at[slot], acc.at[dst_v.at[c]], ssem[slot],
                         add=True)

    def wait_scatter(c, slot):
        pltpu.make_async_copy(rows.at[slot], acc.at[dst_v.at[c]],
                              ssem[slot]).wait()

    for b in range(B - 1):
        gather(b, b)

    def step(c0, carry):
        for b in range(B):
            c = c0 * B + b
            nxt = c + B - 1
            pslot = (b - 1) % B

            @pl.when(jnp.logical_and(nxt < nch, c > 0))
            def _():
                wait_scatter(c - 1, pslot)

            @pl.when(nxt < nch)
            def _():
                gather(nxt, pslot)
            wait_gather(c, b)
            scatter(c, b)
        return carry

    lax.fori_loop(0, nch // B, step, 0)
    for b in range(B):
        c = nch - B + b
        wait_scatter(c, c % B)
    plsc.subcore_barrier()
    pltpu.sync_copy(acc.at[pl.ds(r0, RPT)], out.at[cid, pl.ds(r0, RPT)])


def _make_seg_sum(d, col_split):
    nch = NCH_C if col_split else NCH_E
    mesh = plsc.VectorSubcoreMesh(core_axis_name="c", subcore_axis_name="s")
    scratch = (
        [pltpu.VMEM((nch, CH), jnp.int32),
         pltpu.VMEM((nch, CH), jnp.int32),
         pltpu.VMEM((B, CH, d), jnp.float32),
         pltpu.VMEM_SHARED((N_ACC, d), jnp.float32)]
        + [pltpu.SemaphoreType.DMA] * (2 * B)
    )
    return pl.kernel(
        functools.partial(_seg_body, nch, col_split),
        out_type=jax.ShapeDtypeStruct((NC, N_ACC, d), jnp.float32),
        mesh=mesh, scratch_types=scratch, compiler_params=_SC_PARAMS)


def _deg_body(dsts, ones_in, dout, dst_v, ones_v, dacc, sem):
    cid = lax.axis_index("c")
    sid = lax.axis_index("s")
    wid = cid * NS + sid
    r0 = sid * RPT
    pltpu.sync_copy(ones_in.at[pl.ds(r0, RPT)], dacc.at[pl.ds(r0, RPT)])
    pltpu.sync_copy(dsts.at[wid], dst_v)
    pltpu.sync_copy(ones_in.at[pl.ds(N_ACC, CH)], ones_v)
    plsc.subcore_barrier()

    def step(c0, carry):
        for b in range(8):
            c = c0 * 8 + b
            pltpu.async_copy(ones_v, dacc.at[dst_v.at[c]], sem, add=True)
        for b in range(8):
            c = c0 * 8 + b
            pltpu.make_async_copy(ones_v, dacc.at[dst_v.at[c]], sem).wait()
        return carry

    lax.fori_loop(0, NCH_E // 8, step, 0)
    plsc.subcore_barrier()
    pltpu.sync_copy(dacc.at[pl.ds(r0, RPT)], dout.at[cid, pl.ds(r0, RPT)])


_deg_kernel = pl.kernel(
    _deg_body,
    out_type=jax.ShapeDtypeStruct((NC, N_ACC, 16), jnp.float32),
    mesh=plsc.VectorSubcoreMesh(core_axis_name="c", subcore_axis_name="s"),
    scratch_types=[
        pltpu.VMEM((NCH_E, CH), jnp.int32),
        pltpu.VMEM((CH, 16), jnp.float32),
        pltpu.VMEM_SHARED((N_ACC, 16), jnp.float32),
        pltpu.SemaphoreType.DMA,
    ],
    compiler_params=_SC_PARAMS)


def _mm2_body(x_ref, wl_ref, wr_ref, y_ref, z_ref):
    x = x_ref[...]
    y_ref[0] = jnp.dot(x, wl_ref[0], preferred_element_type=jnp.float32)
    y_ref[1] = jnp.dot(x, wl_ref[1], preferred_element_type=jnp.float32)
    z_ref[...] = jnp.dot(x, wr_ref[...], preferred_element_type=jnp.float32)


def _norm(s_ref, d_ref, z_ref, b_ref):
    s = jnp.concatenate([s_ref[0], s_ref[1]], axis=1)
    deg = d_ref[0, :, 0:1] + d_ref[1, :, 0:1]
    return s / jnp.maximum(deg, 1.0) + b_ref[...] + z_ref[...]


def _mid_body(s_ref, d_ref, z_ref, b_ref, g_ref, be_ref, wl_ref, wr_ref,
              y_ref, zo_ref):
    t = _norm(s_ref, d_ref, z_ref, b_ref)
    m = jnp.mean(t, axis=-1, keepdims=True)
    v = jnp.mean((t - m) ** 2, axis=-1, keepdims=True)
    h = (t - m) / jnp.sqrt(v + 1e-5) * g_ref[...] + be_ref[...]
    h = jnp.maximum(h, 0.0)
    if y_ref.shape[0] == 2:
        y_ref[0] = jnp.dot(h, wl_ref[0], preferred_element_type=jnp.float32)
        y_ref[1] = jnp.dot(h, wl_ref[1], preferred_element_type=jnp.float32)
    else:
        y_ref[...] = jnp.dot(h, wl_ref[...],
                             preferred_element_type=jnp.float32)
    zo_ref[...] = jnp.dot(h, wr_ref[...], preferred_element_type=jnp.float32)


def _fin_body(s_ref, d_ref, z_ref, b_ref, o_ref):
    s = s_ref[0] + s_ref[1]
    deg = d_ref[0, :, 0:1] + d_ref[1, :, 0:1]
    o_ref[...] = s / jnp.maximum(deg, 1.0) + b_ref[...] + z_ref[...]


BLK = 1000
GRID = N // BLK


def _tc_mm2(x, wl2, wr):
    din = x.shape[1]
    dh = wl2.shape[2]
    return pl.pallas_call(
        _mm2_body,
        grid=(GRID,),
        in_specs=[
            pl.BlockSpec((BLK, din), lambda i: (i, 0)),
            pl.BlockSpec((NC, din, dh), lambda i: (0, 0, 0)),
            pl.BlockSpec((din, din), lambda i: (0, 0)),
        ],
        out_specs=[
            pl.BlockSpec((NC, BLK, dh), lambda i: (0, i, 0)),
            pl.BlockSpec((BLK, din), lambda i: (i, 0)),
        ],
        out_shape=[jax.ShapeDtypeStruct((NC, N, dh), jnp.float32),
                   jax.ShapeDtypeStruct((N, din), jnp.float32)],
    )(x, wl2, wr)


def _tc_mid(s, dg, z, b, g, be, wl, wr, split_out):
    din = z.shape[1]
    if split_out:
        dh = wl.shape[2]
        wl_spec = pl.BlockSpec((NC, din, dh), lambda i: (0, 0, 0))
        y_spec = pl.BlockSpec((NC, BLK, dh), lambda i: (0, i, 0))
        y_shape = jax.ShapeDtypeStruct((NC, N, dh), jnp.float32)
        dout = din
    else:
        dout = wl.shape[1]
        wl_spec = pl.BlockSpec((din, dout), lambda i: (0, 0))
        y_spec = pl.BlockSpec((BLK, dout), lambda i: (i, 0))
        y_shape = jax.ShapeDtypeStruct((N, dout), jnp.float32)
    return pl.pallas_call(
        _mid_body,
        grid=(GRID,),
        in_specs=[
            pl.BlockSpec((NC, BLK, din // 2), lambda i: (0, i, 0)),
            pl.BlockSpec((NC, BLK, 16), lambda i: (0, i, 0)),
            pl.BlockSpec((BLK, din), lambda i: (i, 0)),
            pl.BlockSpec((1, din), lambda i: (0, 0)),
            pl.BlockSpec((1, din), lambda i: (0, 0)),
            pl.BlockSpec((1, din), lambda i: (0, 0)),
            wl_spec,
            pl.BlockSpec((din, dout), lambda i: (0, 0)),
        ],
        out_specs=[
            y_spec,
            pl.BlockSpec((BLK, dout), lambda i: (i, 0)),
        ],
        out_shape=[y_shape,
                   jax.ShapeDtypeStruct((N, dout), jnp.float32)],
    )(s, dg, z, b, g, be, wl, wr)


def _tc_fin(s, dg, z, b):
    dout = z.shape[1]
    return pl.pallas_call(
        _fin_body,
        grid=(GRID,),
        in_specs=[
            pl.BlockSpec((NC, BLK, dout), lambda i: (0, i, 0)),
            pl.BlockSpec((NC, BLK, 16), lambda i: (0, i, 0)),
            pl.BlockSpec((BLK, dout), lambda i: (i, 0)),
            pl.BlockSpec((1, dout), lambda i: (0, 0)),
        ],
        out_specs=pl.BlockSpec((BLK, dout), lambda i: (i, 0)),
        out_shape=jax.ShapeDtypeStruct((N, dout), jnp.float32),
    )(s, dg, z, b)


_seg_sum_col = _make_seg_sum(64, True)
_seg_sum_edge = _make_seg_sum(64, False)


def kernel(x, edge_index, W1l, b1l, W1r, W2l, b2l, W2r, W3l, b3l, W3r,
           gamma, beta):
    src = edge_index[0]
    dst = edge_index[1]
    pad = E_PAD - E
    src_p = jnp.concatenate([src, jnp.zeros((pad,), jnp.int32)])
    dst_p = jnp.concatenate([dst, jnp.full((pad,), N, jnp.int32)])
    src_e = src_p.reshape(NW, NCH_E, CH)
    dst_e = dst_p.reshape(NW, NCH_E, CH)
    src_c = src_p.reshape(NS, NCH_C, CH)
    dst_c = dst_p.reshape(NS, NCH_C, CH)
    z64 = jnp.zeros((N_ACC, 64), jnp.float32)
    ones_deg = jnp.concatenate([jnp.zeros((N_ACC, 16), jnp.float32),
                                jnp.ones((CH, 16), jnp.float32)])
    b1 = b1l.reshape(1, -1)
    b2 = b2l.reshape(1, -1)
    b3 = b3l.reshape(1, -1)
    g = gamma.reshape(1, -1)
    be = beta.reshape(1, -1)
    w1l2 = W1l.reshape(128, NC, 64).transpose(1, 0, 2)
    w2l2 = W2l.reshape(128, NC, 64).transpose(1, 0, 2)

    dg = _deg_kernel(dst_e, ones_deg)
    y1, zz1 = _tc_mm2(x, w1l2, W1r)
    s1 = _seg_sum_col(y1, src_c, dst_c, z64)
    y2, zz2 = _tc_mid(s1, dg, zz1, b1, g, be, w2l2, W2r, True)
    s2 = _seg_sum_col(y2, src_c, dst_c, z64)
    y3, zz3 = _tc_mid(s2, dg, zz2, b2, g, be, W3l, W3r, False)
    s3 = _seg_sum_edge(y3, src_e, dst_e, z64)
    return _tc_fin(s3, dg, zz3, b3)


# speedup vs baseline: 5.5802x; 1.0012x over previous
import functools

import jax
import jax.numpy as jnp
from jax import lax
from jax.experimental import pallas as pl
from jax.experimental.pallas import tpu as pltpu
from jax.experimental.pallas import tpu_sc as plsc

N = 10000
E = 320000
NC, NS = 2, 16
NW = NC * NS
CH = 128
NCH_E = 80
NCH_C = 160
E_PAD = NW * NCH_E * CH
N_ACC = 10112
RPT = N_ACC // NS
B = 4

_SC_PARAMS = pltpu.CompilerParams(use_tc_tiling_on_sc=False)


def _seg_body(nch, col_split, y, srcs, dsts, zeros, out,
              src_v, dst_v, rows, acc, *sems):
    gsem = sems[:B]
    ssem = sems[B:]
    cid = lax.axis_index("c")
    sid = lax.axis_index("s")
    r0 = sid * RPT
    pltpu.sync_copy(zeros.at[pl.ds(r0, RPT)], acc.at[pl.ds(r0, RPT)])
    if col_split:
        table = y.at[cid]
        pltpu.sync_copy(srcs.at[sid], src_v)
        pltpu.sync_copy(dsts.at[sid], dst_v)
    else:
        table = y
        wid = cid * NS + sid
        pltpu.sync_copy(srcs.at[wid], src_v)
        pltpu.sync_copy(dsts.at[wid], dst_v)
    plsc.subcore_barrier()

    def gather(c, slot):
        pltpu.async_copy(table.at[src_v.at[c]], rows.at[slot], gsem[slot])

    def wait_gather(c, slot):
        pltpu.make_async_copy(table.at[src_v.at[c]], rows.at[slot],
                              gsem[slot]).wait()

    def scatter(c, slot):
        pltpu.async_copy(rows.at[slot], acc.at[dst_v.at[c]], ssem[slot],
                         add=True)

    def wait_scatter(c, slot):
        pltpu.make_async_copy(rows.at[slot], acc.at[dst_v.at[c]],
                              ssem[slot]).wait()

    for b in range(B - 1):
        gather(b, b)

    def step(c0, carry):
        for b in range(B):
            c = c0 * B + b
            nxt = c + B - 1
            pslot = (b - 1) % B

            @pl.when(jnp.logical_and(nxt < nch, c > 0))
            def _():
                wait_scatter(c - 1, pslot)

            @pl.when(nxt < nch)
            def _():
                gather(nxt, pslot)
            wait_gather(c, b)
            scatter(c, b)
        return carry

    lax.fori_loop(0, nch // B, step, 0)
    for b in range(B):
        c = nch - B + b
        wait_scatter(c, c % B)
    plsc.subcore_barrier()
    pltpu.sync_copy(acc.at[pl.ds(r0, RPT)], out.at[cid, pl.ds(r0, RPT)])


def _make_seg_sum(d, col_split, chw):
    ept = (NCH_C if col_split else NCH_E) * CH
    nch = ept // chw
    mesh = plsc.VectorSubcoreMesh(core_axis_name="c", subcore_axis_name="s")
    scratch = (
        [pltpu.VMEM((nch, chw), jnp.int32),
         pltpu.VMEM((nch, chw), jnp.int32),
         pltpu.VMEM((B, chw, d), jnp.float32),
         pltpu.VMEM_SHARED((N_ACC, d), jnp.float32)]
        + [pltpu.SemaphoreType.DMA] * (2 * B)
    )
    return pl.kernel(
        functools.partial(_seg_body, nch, col_split),
        out_type=jax.ShapeDtypeStruct((NC, N_ACC, d), jnp.float32),
        mesh=mesh, scratch_types=scratch, compiler_params=_SC_PARAMS)


def _deg_body(dsts, ones_in, dout, dst_v, ones_v, dacc, sem):
    cid = lax.axis_index("c")
    sid = lax.axis_index("s")
    wid = cid * NS + sid
    r0 = sid * RPT
    pltpu.sync_copy(ones_in.at[pl.ds(r0, RPT)], dacc.at[pl.ds(r0, RPT)])
    pltpu.sync_copy(dsts.at[wid], dst_v)
    pltpu.sync_copy(ones_in.at[pl.ds(N_ACC, CH)], ones_v)
    plsc.subcore_barrier()

    def step(c0, carry):
        for b in range(8):
            c = c0 * 8 + b
            pltpu.async_copy(ones_v, dacc.at[dst_v.at[c]], sem, add=True)
        for b in range(8):
            c = c0 * 8 + b
            pltpu.make_async_copy(ones_v, dacc.at[dst_v.at[c]], sem).wait()
        return carry

    lax.fori_loop(0, NCH_E // 8, step, 0)
    plsc.subcore_barrier()
    pltpu.sync_copy(dacc.at[pl.ds(r0, RPT)], dout.at[cid, pl.ds(r0, RPT)])


_deg_kernel = pl.kernel(
    _deg_body,
    out_type=jax.ShapeDtypeStruct((NC, N_ACC, 16), jnp.float32),
    mesh=plsc.VectorSubcoreMesh(core_axis_name="c", subcore_axis_name="s"),
    scratch_types=[
        pltpu.VMEM((NCH_E, CH), jnp.int32),
        pltpu.VMEM((CH, 16), jnp.float32),
        pltpu.VMEM_SHARED((N_ACC, 16), jnp.float32),
        pltpu.SemaphoreType.DMA,
    ],
    compiler_params=_SC_PARAMS)


def _mm2_body(x_ref, wl_ref, wr_ref, y_ref, z_ref):
    x = x_ref[...]
    y_ref[0] = jnp.dot(x, wl_ref[0], preferred_element_type=jnp.float32)
    y_ref[1] = jnp.dot(x, wl_ref[1], preferred_element_type=jnp.float32)
    z_ref[...] = jnp.dot(x, wr_ref[...], preferred_element_type=jnp.float32)


def _norm(s_ref, d_ref, z_ref, b_ref):
    s = jnp.concatenate([s_ref[0], s_ref[1]], axis=1)
    deg = d_ref[0, :, 0:1] + d_ref[1, :, 0:1]
    return s / jnp.maximum(deg, 1.0) + b_ref[...] + z_ref[...]


def _mid_body(s_ref, d_ref, z_ref, b_ref, g_ref, be_ref, wl_ref, wr_ref,
              y_ref, zo_ref):
    t = _norm(s_ref, d_ref, z_ref, b_ref)
    m = jnp.mean(t, axis=-1, keepdims=True)
    v = jnp.mean((t - m) ** 2, axis=-1, keepdims=True)
    h = (t - m) / jnp.sqrt(v + 1e-5) * g_ref[...] + be_ref[...]
    h = jnp.maximum(h, 0.0)
    if y_ref.shape[0] == 2:
        y_ref[0] = jnp.dot(h, wl_ref[0], preferred_element_type=jnp.float32)
        y_ref[1] = jnp.dot(h, wl_ref[1], preferred_element_type=jnp.float32)
    else:
        y_ref[...] = jnp.dot(h, wl_ref[...],
                             preferred_element_type=jnp.float32)
    zo_ref[...] = jnp.dot(h, wr_ref[...], preferred_element_type=jnp.float32)


def _fin_body(s_ref, d_ref, z_ref, b_ref, o_ref):
    s = s_ref[0] + s_ref[1]
    deg = d_ref[0, :, 0:1] + d_ref[1, :, 0:1]
    o_ref[...] = s / jnp.maximum(deg, 1.0) + b_ref[...] + z_ref[...]


BLK = 1000
GRID = N // BLK


def _tc_mm2(x, wl2, wr):
    din = x.shape[1]
    dh = wl2.shape[2]
    return pl.pallas_call(
        _mm2_body,
        grid=(GRID,),
        in_specs=[
            pl.BlockSpec((BLK, din), lambda i: (i, 0)),
            pl.BlockSpec((NC, din, dh), lambda i: (0, 0, 0)),
            pl.BlockSpec((din, din), lambda i: (0, 0)),
        ],
        out_specs=[
            pl.BlockSpec((NC, BLK, dh), lambda i: (0, i, 0)),
            pl.BlockSpec((BLK, din), lambda i: (i, 0)),
        ],
        out_shape=[jax.ShapeDtypeStruct((NC, N, dh), jnp.float32),
                   jax.ShapeDtypeStruct((N, din), jnp.float32)],
    )(x, wl2, wr)


def _tc_mid(s, dg, z, b, g, be, wl, wr, split_out):
    din = z.shape[1]
    if split_out:
        dh = wl.shape[2]
        wl_spec = pl.BlockSpec((NC, din, dh), lambda i: (0, 0, 0))
        y_spec = pl.BlockSpec((NC, BLK, dh), lambda i: (0, i, 0))
        y_shape = jax.ShapeDtypeStruct((NC, N, dh), jnp.float32)
        dout = din
    else:
        dout = wl.shape[1]
        wl_spec = pl.BlockSpec((din, dout), lambda i: (0, 0))
        y_spec = pl.BlockSpec((BLK, dout), lambda i: (i, 0))
        y_shape = jax.ShapeDtypeStruct((N, dout), jnp.float32)
    return pl.pallas_call(
        _mid_body,
        grid=(GRID,),
        in_specs=[
            pl.BlockSpec((NC, BLK, din // 2), lambda i: (0, i, 0)),
            pl.BlockSpec((NC, BLK, 16), lambda i: (0, i, 0)),
            pl.BlockSpec((BLK, din), lambda i: (i, 0)),
            pl.BlockSpec((1, din), lambda i: (0, 0)),
            pl.BlockSpec((1, din), lambda i: (0, 0)),
            pl.BlockSpec((1, din), lambda i: (0, 0)),
            wl_spec,
            pl.BlockSpec((din, dout), lambda i: (0, 0)),
        ],
        out_specs=[
            y_spec,
            pl.BlockSpec((BLK, dout), lambda i: (i, 0)),
        ],
        out_shape=[y_shape,
                   jax.ShapeDtypeStruct((N, dout), jnp.float32)],
    )(s, dg, z, b, g, be, wl, wr)


def _tc_fin(s, dg, z, b):
    dout = z.shape[1]
    return pl.pallas_call(
        _fin_body,
        grid=(GRID,),
        in_specs=[
            pl.BlockSpec((NC, BLK, dout), lambda i: (0, i, 0)),
            pl.BlockSpec((NC, BLK, 16), lambda i: (0, i, 0)),
            pl.BlockSpec((BLK, dout), lambda i: (i, 0)),
            pl.BlockSpec((1, dout), lambda i: (0, 0)),
        ],
        out_specs=pl.BlockSpec((BLK, dout), lambda i: (i, 0)),
        out_shape=jax.ShapeDtypeStruct((N, dout), jnp.float32),
    )(s, dg, z, b)


_seg_sum_col = _make_seg_sum(64, True, 128)
_seg_sum_edge = _make_seg_sum(64, False, 256)


def kernel(x, edge_index, W1l, b1l, W1r, W2l, b2l, W2r, W3l, b3l, W3r,
           gamma, beta):
    src = edge_index[0]
    dst = edge_index[1]
    pad = E_PAD - E
    src_p = jnp.concatenate([src, jnp.zeros((pad,), jnp.int32)])
    dst_p = jnp.concatenate([dst, jnp.full((pad,), N, jnp.int32)])
    src_e = src_p.reshape(NW, NCH_E * CH // 256, 256)
    dst_e = dst_p.reshape(NW, NCH_E * CH // 256, 256)
    dst_d = dst_p.reshape(NW, NCH_E, CH)
    src_c = src_p.reshape(NS, NCH_C, CH)
    dst_c = dst_p.reshape(NS, NCH_C, CH)
    z64 = jnp.zeros((N_ACC, 64), jnp.float32)
    ones_deg = jnp.concatenate([jnp.zeros((N_ACC, 16), jnp.float32),
                                jnp.ones((CH, 16), jnp.float32)])
    b1 = b1l.reshape(1, -1)
    b2 = b2l.reshape(1, -1)
    b3 = b3l.reshape(1, -1)
    g = gamma.reshape(1, -1)
    be = beta.reshape(1, -1)
    w1l2 = W1l.reshape(128, NC, 64).transpose(1, 0, 2)
    w2l2 = W2l.reshape(128, NC, 64).transpose(1, 0, 2)

    dg = _deg_kernel(dst_d, ones_deg)
    y1, zz1 = _tc_mm2(x, w1l2, W1r)
    s1 = _seg_sum_col(y1, src_c, dst_c, z64)
    y2, zz2 = _tc_mid(s1, dg, zz1, b1, g, be, w2l2, W2r, True)
    s2 = _seg_sum_col(y2, src_c, dst_c, z64)
    y3, zz3 = _tc_mid(s2, dg, zz2, b2, g, be, W3l, W3r, False)
    s3 = _seg_sum_edge(y3, src_e, dst_e, z64)
    return _tc_fin(s3, dg, zz3, b3)


# speedup vs baseline: 5.5837x; 1.0006x over previous
import functools

import jax
import jax.numpy as jnp
from jax import lax
from jax.experimental import pallas as pl
from jax.experimental.pallas import tpu as pltpu
from jax.experimental.pallas import tpu_sc as plsc

N = 10000
E = 320000
NC, NS = 2, 16
NW = NC * NS
CH = 128
NCH_E = 80
NCH_C = 160
E_PAD = NW * NCH_E * CH
N_ACC = 10112
RPT = N_ACC // NS

_SC_PARAMS = pltpu.CompilerParams(use_tc_tiling_on_sc=False)


def _seg_body(nch, bb, pp, col_split, y, srcs, dsts, zeros, out,
              src_v, dst_v, rows, acc, *sems):
    gsem = sems[:bb]
    ssem = sems[bb:]
    cid = lax.axis_index("c")
    sid = lax.axis_index("s")
    r0 = sid * RPT
    pltpu.sync_copy(zeros.at[pl.ds(r0, RPT)], acc.at[pl.ds(r0, RPT)])
    if col_split:
        table = y.at[cid]
        pltpu.sync_copy(srcs.at[sid], src_v)
        pltpu.sync_copy(dsts.at[sid], dst_v)
    else:
        table = y
        wid = cid * NS + sid
        pltpu.sync_copy(srcs.at[wid], src_v)
        pltpu.sync_copy(dsts.at[wid], dst_v)
    plsc.subcore_barrier()

    def gather(c, slot):
        pltpu.async_copy(table.at[src_v.at[c]], rows.at[slot], gsem[slot])

    def wait_gather(c, slot):
        pltpu.make_async_copy(table.at[src_v.at[c]], rows.at[slot],
                              gsem[slot]).wait()

    def scatter(c, slot):
        pltpu.async_copy(rows.at[slot], acc.at[dst_v.at[c]], ssem[slot],
                         add=True)

    def wait_scatter(c, slot):
        pltpu.make_async_copy(rows.at[slot], acc.at[dst_v.at[c]],
                              ssem[slot]).wait()

    for b in range(pp):
        gather(b, b)

    def step(c0, carry):
        for b in range(bb):
            c = c0 * bb + b
            nxt = c + pp
            nslot = (b + pp) % bb

            @pl.when(nxt < nch)
            def _():
                @pl.when(nxt >= bb)
                def _():
                    wait_scatter(nxt - bb, nslot)
                gather(nxt, nslot)
            wait_gather(c, b)
            scatter(c, b)
        return carry

    lax.fori_loop(0, nch // bb, step, 0)
    for b in range(bb):
        c = nch - bb + b
        wait_scatter(c, c % bb)
    plsc.subcore_barrier()
    pltpu.sync_copy(acc.at[pl.ds(r0, RPT)], out.at[cid, pl.ds(r0, RPT)])


def _make_seg_sum(d, col_split, chw, bb, pp):
    ept = (NCH_C if col_split else NCH_E) * CH
    nch = ept // chw
    mesh = plsc.VectorSubcoreMesh(core_axis_name="c", subcore_axis_name="s")
    scratch = (
        [pltpu.VMEM((nch, chw), jnp.int32),
         pltpu.VMEM((nch, chw), jnp.int32),
         pltpu.VMEM((bb, chw, d), jnp.float32),
         pltpu.VMEM_SHARED((N_ACC, d), jnp.float32)]
        + [pltpu.SemaphoreType.DMA] * (2 * bb)
    )
    return pl.kernel(
        functools.partial(_seg_body, nch, bb, pp, col_split),
        out_type=jax.ShapeDtypeStruct((NC, N_ACC, d), jnp.float32),
        mesh=mesh, scratch_types=scratch, compiler_params=_SC_PARAMS)


def _deg_body(dsts, ones_in, dout, dst_v, ones_v, dacc, sem):
    cid = lax.axis_index("c")
    sid = lax.axis_index("s")
    wid = cid * NS + sid
    r0 = sid * RPT
    pltpu.sync_copy(ones_in.at[pl.ds(r0, RPT)], dacc.at[pl.ds(r0, RPT)])
    pltpu.sync_copy(dsts.at[wid], dst_v)
    pltpu.sync_copy(ones_in.at[pl.ds(N_ACC, CH)], ones_v)
    plsc.subcore_barrier()

    def step(c0, carry):
        for b in range(8):
            c = c0 * 8 + b
            pltpu.async_copy(ones_v, dacc.at[dst_v.at[c]], sem, add=True)
        for b in range(8):
            c = c0 * 8 + b
            pltpu.make_async_copy(ones_v, dacc.at[dst_v.at[c]], sem).wait()
        return carry

    lax.fori_loop(0, NCH_E // 8, step, 0)
    plsc.subcore_barrier()
    pltpu.sync_copy(dacc.at[pl.ds(r0, RPT)], dout.at[cid, pl.ds(r0, RPT)])


_deg_kernel = pl.kernel(
    _deg_body,
    out_type=jax.ShapeDtypeStruct((NC, N_ACC, 16), jnp.float32),
    mesh=plsc.VectorSubcoreMesh(core_axis_name="c", subcore_axis_name="s"),
    scratch_types=[
        pltpu.VMEM((NCH_E, CH), jnp.int32),
        pltpu.VMEM((CH, 16), jnp.float32),
        pltpu.VMEM_SHARED((N_ACC, 16), jnp.float32),
        pltpu.SemaphoreType.DMA,
    ],
    compiler_params=_SC_PARAMS)


def _mm2_body(x_ref, wl_ref, wr_ref, y_ref, z_ref):
    x = x_ref[...]
    y_ref[0] = jnp.dot(x, wl_ref[0], preferred_element_type=jnp.float32)
    y_ref[1] = jnp.dot(x, wl_ref[1], preferred_element_type=jnp.float32)
    z_ref[...] = jnp.dot(x, wr_ref[...], preferred_element_type=jnp.float32)


def _norm(s_ref, d_ref, z_ref, b_ref):
    s = jnp.concatenate([s_ref[0], s_ref[1]], axis=1)
    deg = d_ref[0, :, 0:1] + d_ref[1, :, 0:1]
    return s / jnp.maximum(deg, 1.0) + b_ref[...] + z_ref[...]


def _mid_body(s_ref, d_ref, z_ref, b_ref, g_ref, be_ref, wl_ref, wr_ref,
              y_ref, zo_ref):
    t = _norm(s_ref, d_ref, z_ref, b_ref)
    m = jnp.mean(t, axis=-1, keepdims=True)
    v = jnp.mean((t - m) ** 2, axis=-1, keepdims=True)
    h = (t - m) / jnp.sqrt(v + 1e-5) * g_ref[...] + be_ref[...]
    h = jnp.maximum(h, 0.0)
    if y_ref.shape[0] == 2:
        y_ref[0] = jnp.dot(h, wl_ref[0], preferred_element_type=jnp.float32)
        y_ref[1] = jnp.dot(h, wl_ref[1], preferred_element_type=jnp.float32)
    else:
        y_ref[...] = jnp.dot(h, wl_ref[...],
                             preferred_element_type=jnp.float32)
    zo_ref[...] = jnp.dot(h, wr_ref[...], preferred_element_type=jnp.float32)


def _fin_body(s_ref, d_ref, z_ref, b_ref, o_ref):
    s = s_ref[0] + s_ref[1]
    deg = d_ref[0, :, 0:1] + d_ref[1, :, 0:1]
    o_ref[...] = s / jnp.maximum(deg, 1.0) + b_ref[...] + z_ref[...]


BLK = 1000
GRID = N // BLK


def _tc_mm2(x, wl2, wr):
    din = x.shape[1]
    dh = wl2.shape[2]
    return pl.pallas_call(
        _mm2_body,
        grid=(GRID,),
        in_specs=[
            pl.BlockSpec((BLK, din), lambda i: (i, 0)),
            pl.BlockSpec((NC, din, dh), lambda i: (0, 0, 0)),
            pl.BlockSpec((din, din), lambda i: (0, 0)),
        ],
        out_specs=[
            pl.BlockSpec((NC, BLK, dh), lambda i: (0, i, 0)),
            pl.BlockSpec((BLK, din), lambda i: (i, 0)),
        ],
        out_shape=[jax.ShapeDtypeStruct((NC, N, dh), jnp.float32),
                   jax.ShapeDtypeStruct((N, din), jnp.float32)],
    )(x, wl2, wr)


def _tc_mid(s, dg, z, b, g, be, wl, wr, split_out):
    din = z.shape[1]
    if split_out:
        dh = wl.shape[2]
        wl_spec = pl.BlockSpec((NC, din, dh), lambda i: (0, 0, 0))
        y_spec = pl.BlockSpec((NC, BLK, dh), lambda i: (0, i, 0))
        y_shape = jax.ShapeDtypeStruct((NC, N, dh), jnp.float32)
        dout = din
    else:
        dout = wl.shape[1]
        wl_spec = pl.BlockSpec((din, dout), lambda i: (0, 0))
        y_spec = pl.BlockSpec((BLK, dout), lambda i: (i, 0))
        y_shape = jax.ShapeDtypeStruct((N, dout), jnp.float32)
    return pl.pallas_call(
        _mid_body,
        grid=(GRID,),
        in_specs=[
            pl.BlockSpec((NC, BLK, din // 2), lambda i: (0, i, 0)),
            pl.BlockSpec((NC, BLK, 16), lambda i: (0, i, 0)),
            pl.BlockSpec((BLK, din), lambda i: (i, 0)),
            pl.BlockSpec((1, din), lambda i: (0, 0)),
            pl.BlockSpec((1, din), lambda i: (0, 0)),
            pl.BlockSpec((1, din), lambda i: (0, 0)),
            wl_spec,
            pl.BlockSpec((din, dout), lambda i: (0, 0)),
        ],
        out_specs=[
            y_spec,
            pl.BlockSpec((BLK, dout), lambda i: (i, 0)),
        ],
        out_shape=[y_shape,
                   jax.ShapeDtypeStruct((N, dout), jnp.float32)],
    )(s, dg, z, b, g, be, wl, wr)


def _tc_fin(s, dg, z, b):
    dout = z.shape[1]
    return pl.pallas_call(
        _fin_body,
        grid=(GRID,),
        in_specs=[
            pl.BlockSpec((NC, BLK, dout), lambda i: (0, i, 0)),
            pl.BlockSpec((NC, BLK, 16), lambda i: (0, i, 0)),
            pl.BlockSpec((BLK, dout), lambda i: (i, 0)),
            pl.BlockSpec((1, dout), lambda i: (0, 0)),
        ],
        out_specs=pl.BlockSpec((BLK, dout), lambda i: (i, 0)),
        out_shape=jax.ShapeDtypeStruct((N, dout), jnp.float32),
    )(s, dg, z, b)


_seg_sum_col = _make_seg_sum(64, True, 128, 5, 3)
_seg_sum_edge = _make_seg_sum(64, False, 128, 8, 5)


def kernel(x, edge_index, W1l, b1l, W1r, W2l, b2l, W2r, W3l, b3l, W3r,
           gamma, beta):
    src = edge_index[0]
    dst = edge_index[1]
    pad = E_PAD - E
    src_p = jnp.concatenate([src, jnp.zeros((pad,), jnp.int32)])
    dst_p = jnp.concatenate([dst, jnp.full((pad,), N, jnp.int32)])
    src_e = src_p.reshape(NW, NCH_E, CH)
    dst_e = dst_p.reshape(NW, NCH_E, CH)
    src_c = src_p.reshape(NS, NCH_C, CH)
    dst_c = dst_p.reshape(NS, NCH_C, CH)
    z64 = jnp.zeros((N_ACC, 64), jnp.float32)
    ones_deg = jnp.concatenate([jnp.zeros((N_ACC, 16), jnp.float32),
                                jnp.ones((CH, 16), jnp.float32)])
    b1 = b1l.reshape(1, -1)
    b2 = b2l.reshape(1, -1)
    b3 = b3l.reshape(1, -1)
    g = gamma.reshape(1, -1)
    be = beta.reshape(1, -1)
    w1l2 = W1l.reshape(128, NC, 64).transpose(1, 0, 2)
    w2l2 = W2l.reshape(128, NC, 64).transpose(1, 0, 2)

    dg = _deg_kernel(dst_e, ones_deg)
    y1, zz1 = _tc_mm2(x, w1l2, W1r)
    s1 = _seg_sum_col(y1, src_c, dst_c, z64)
    y2, zz2 = _tc_mid(s1, dg, zz1, b1, g, be, w2l2, W2r, True)
    s2 = _seg_sum_col(y2, src_c, dst_c, z64)
    y3, zz3 = _tc_mid(s2, dg, zz2, b2, g, be, W3l, W3r, False)
    s3 = _seg_sum_edge(y3, src_e, dst_e, z64)
    return _tc_fin(s3, dg, zz3, b3)
